# Initial kernel scaffold; baseline (speedup 1.0000x reference)
import jax
import jax.numpy as jnp
from jax.experimental import pallas as pl


def kernel(c, pseudo_label):
    raise NotImplementedError("write your pallas kernel here")


# speedup vs baseline: 1.8355x; 1.8355x over previous
import jax
import jax.numpy as jnp
from jax.experimental import pallas as pl
from jax.experimental.pallas import tpu as pltpu

BATCH = 16384
K = 1000
BR = 512
NB = BATCH // BR


def _body(lbl_ref, c_ref, loss_ref, cnt_s, nls_s):
    k = pl.program_id(0)

    @pl.when(k == 0)
    def _init():
        cnt_s[...] = jnp.zeros_like(cnt_s)
        nls_s[...] = jnp.zeros_like(nls_s)

    cb = c_ref[...]
    lbl = lbl_ref[...]
    m = jnp.max(cb, axis=1, keepdims=True)
    s = jnp.sum(jnp.exp(cb - m), axis=1, keepdims=True)
    lse = m + jnp.log(s)

    onehot = jax.lax.broadcasted_iota(jnp.int32, (BR, K), 1) == lbl
    g = jnp.sum(jnp.where(onehot, cb, 0.0), axis=1, keepdims=True)
    nll = lse - g

    cnt_s[...] += jnp.sum(onehot.astype(jnp.float32), axis=0, keepdims=True)
    nls_s[...] += jnp.sum(jnp.where(onehot, nll, 0.0), axis=0, keepdims=True)

    @pl.when(k == NB - 1)
    def _final():
        cnt = cnt_s[...]
        nls = nls_s[...]
        present = cnt > 0.0
        per_class = jnp.where(present, nls / jnp.where(present, cnt, 1.0), 0.0)
        num = jnp.sum(per_class, keepdims=True)
        den = jnp.sum(present.astype(jnp.float32), keepdims=True)
        loss_ref[...] = num / den


def kernel(c, pseudo_label):
    lbl = pseudo_label.astype(jnp.int32).reshape(BATCH, 1)
    out = pl.pallas_call(
        _body,
        grid=(NB,),
        in_specs=[
            pl.BlockSpec((BR, 1), lambda k: (k, 0)),
            pl.BlockSpec((BR, K), lambda k: (k, 0)),
        ],
        out_specs=pl.BlockSpec((1, 1), lambda k: (0, 0)),
        out_shape=jax.ShapeDtypeStruct((1, 1), jnp.float32),
        scratch_shapes=[
            pltpu.VMEM((1, K), jnp.float32),
            pltpu.VMEM((1, K), jnp.float32),
        ],
    )(lbl, c)
    return out[0, 0]
